# baseline (speedup 1.0000x reference)
import functools

import jax
import jax.numpy as jnp
from jax import lax
from jax.experimental import pallas as pl
from jax.experimental.pallas import tpu as pltpu
from jax.experimental.pallas import tpu_sc as plsc

B = 16384
F = 26
NC = 2
NS = 16
NW = NC * NS
BPW = B // NW
CHUNK = BPW * F
FIELD = 40000


def _sc_kernel(x_hbm, bias_hbm, tbl_hbm, out_hbm,
               xbuf_v, idx_v, vals_v, bias_v, obuf_v, sem):
    wid = lax.axis_index("c") * NS + lax.axis_index("s")

    pltpu.sync_copy(x_hbm.at[pl.ds(wid * CHUNK, CHUNK)], xbuf_v)
    pltpu.sync_copy(bias_hbm, bias_v)

    iota = lax.iota(jnp.int32, 16)

    def build_idx(c, _):
        p0 = (c * 16 + iota) * F
        for f in range(F):
            xv = plsc.load_gather(xbuf_v, [p0 + f])
            idx_v[pl.ds(f * BPW + c * 16, 16)] = xv + (f * FIELD)
        return _
    lax.fori_loop(0, BPW // 16, build_idx, 0)

    pltpu.async_copy(tbl_hbm.at[0].at[idx_v], vals_v, sem).wait()

    bias16 = bias_v[...]

    def reduce16(c, _):
        r = c * 16
        acc = vals_v[pl.ds(r, 16)]
        for f in range(1, F):
            acc = acc + vals_v[pl.ds(f * BPW + r, 16)]
        obuf_v[pl.ds(r, 16)] = acc + bias16
        return _
    lax.fori_loop(0, BPW // 16, reduce16, 0)

    pltpu.sync_copy(obuf_v, out_hbm.at[pl.ds(wid * BPW, BPW)])


@jax.jit
def kernel(x, table, bias):
    x_flat = x.astype(jnp.int32).reshape(-1)
    tbl_row = table.reshape(1, -1)
    bias16 = jnp.broadcast_to(bias.astype(jnp.float32), (16,))

    run = functools.partial(
        pl.kernel,
        mesh=plsc.VectorSubcoreMesh(core_axis_name="c", subcore_axis_name="s"),
        out_type=jax.ShapeDtypeStruct((B,), jnp.float32),
        compiler_params=pltpu.CompilerParams(
            needs_layout_passes=False,
            use_tc_tiling_on_sc=False,
        ),
        scratch_types=[
            pltpu.VMEM((CHUNK,), jnp.int32),
            pltpu.VMEM((CHUNK,), jnp.int32),
            pltpu.VMEM((CHUNK,), jnp.float32),
            pltpu.VMEM((16,), jnp.float32),
            pltpu.VMEM((BPW,), jnp.float32),
            pltpu.SemaphoreType.DMA,
        ],
    )(_sc_kernel)

    out = run(x_flat, bias16, tbl_row)
    return out.reshape(B, 1)


# speedup vs baseline: 1.2189x; 1.2189x over previous
import functools

import jax
import jax.numpy as jnp
from jax import lax
from jax.experimental import pallas as pl
from jax.experimental.pallas import tpu as pltpu
from jax.experimental.pallas import tpu_sc as plsc

B = 16384
F = 26
NC = 2
NS = 16
NW = NC * NS
BPW = B // NW
CHUNK = BPW * F
FIELD = 40000


def _sc_kernel(xt_hbm, bias_hbm, tbl_hbm, out_hbm,
               idx_v, vals_v, bias_v, obuf_v, sem, gsem):
    wid = lax.axis_index("c") * NS + lax.axis_index("s")

    descs = [
        pltpu.async_copy(
            xt_hbm.at[pl.ds(f * B + wid * BPW, BPW)],
            idx_v.at[pl.ds(f * BPW, BPW)],
            sem,
        )
        for f in range(F)
    ]
    pltpu.sync_copy(bias_hbm, bias_v)
    for d in descs:
        d.wait()

    pltpu.async_copy(tbl_hbm.at[idx_v], vals_v, gsem).wait()

    bias16 = bias_v[...]

    def reduce16(c, _):
        r = c * 16
        acc = vals_v[pl.ds(r, 16)]
        for f in range(1, F):
            acc = acc + vals_v[pl.ds(f * BPW + r, 16)]
        obuf_v[pl.ds(r, 16)] = acc + bias16
        return _
    lax.fori_loop(0, BPW // 16, reduce16, 0)

    pltpu.sync_copy(obuf_v, out_hbm.at[pl.ds(wid * BPW, BPW)])


@jax.jit
def kernel(x, table, bias):
    offsets = jnp.arange(F, dtype=jnp.int32) * FIELD
    xt_flat = (x.astype(jnp.int32) + offsets[None, :]).T.reshape(-1)
    tbl_flat = table.reshape(-1)
    bias16 = jnp.broadcast_to(bias.astype(jnp.float32), (16,))

    run = functools.partial(
        pl.kernel,
        mesh=plsc.VectorSubcoreMesh(core_axis_name="c", subcore_axis_name="s"),
        out_type=jax.ShapeDtypeStruct((B,), jnp.float32),
        compiler_params=pltpu.CompilerParams(needs_layout_passes=False),
        scratch_types=[
            pltpu.VMEM((CHUNK,), jnp.int32),
            pltpu.VMEM((CHUNK,), jnp.float32),
            pltpu.VMEM((16,), jnp.float32),
            pltpu.VMEM((BPW,), jnp.float32),
            pltpu.SemaphoreType.DMA,
            pltpu.SemaphoreType.DMA,
        ],
    )(_sc_kernel)

    out = run(xt_flat, bias16, tbl_flat)
    return out.reshape(B, 1)


# speedup vs baseline: 1.2695x; 1.0415x over previous
import functools

import jax
import jax.numpy as jnp
from jax import lax
from jax.experimental import pallas as pl
from jax.experimental.pallas import tpu as pltpu
from jax.experimental.pallas import tpu_sc as plsc

B = 16384
F = 26
NC = 2
NS = 16
NW = NC * NS
BPW = B // NW
CHUNK = BPW * F
FIELD = 40000


def _sc_kernel(xt_hbm, bias_hbm, tbl_hbm, out_hbm,
               idx_v, vals_v, bias_v, obuf_v, tstage_v, tbl_sp, sem, tsem, gsem):
    sid = lax.axis_index("s")
    wid = lax.axis_index("c") * NS + sid

    descs = [
        pltpu.async_copy(
            xt_hbm.at[pl.ds(f * B + wid * BPW, BPW)],
            idx_v.at[pl.ds(f * BPW, BPW)],
            sem,
        )
        for f in range(F)
    ]
    pltpu.sync_copy(bias_hbm, bias_v)
    tpw = 1040000 // NS
    tchunk = tpw // 5

    def stage_tbl(k, _):
        base = sid * tpw + k * tchunk
        pltpu.sync_copy(tbl_hbm.at[pl.ds(base, tchunk)], tstage_v)
        pltpu.sync_copy(tstage_v, tbl_sp.at[pl.ds(base, tchunk)])
        return _
    lax.fori_loop(0, 5, stage_tbl, 0)
    for d in descs:
        d.wait()
    plsc.subcore_barrier()

    pltpu.async_copy(tbl_sp.at[idx_v], vals_v, gsem).wait()

    bias16 = bias_v[...]

    def reduce16(c, _):
        r = c * 16
        acc = vals_v[pl.ds(r, 16)]
        for f in range(1, F):
            acc = acc + vals_v[pl.ds(f * BPW + r, 16)]
        obuf_v[pl.ds(r, 16)] = acc + bias16
        return _
    lax.fori_loop(0, BPW // 16, reduce16, 0)

    pltpu.sync_copy(obuf_v, out_hbm.at[pl.ds(wid * BPW, BPW)])


@jax.jit
def kernel(x, table, bias):
    offsets = jnp.arange(F, dtype=jnp.int32) * FIELD
    xt_flat = (x.astype(jnp.int32) + offsets[None, :]).T.reshape(-1)
    tbl_flat = table.reshape(-1)
    bias16 = jnp.broadcast_to(bias.astype(jnp.float32), (16,))

    run = functools.partial(
        pl.kernel,
        mesh=plsc.VectorSubcoreMesh(core_axis_name="c", subcore_axis_name="s"),
        out_type=jax.ShapeDtypeStruct((B,), jnp.float32),
        compiler_params=pltpu.CompilerParams(needs_layout_passes=False),
        scratch_types=[
            pltpu.VMEM((CHUNK,), jnp.int32),
            pltpu.VMEM((CHUNK,), jnp.float32),
            pltpu.VMEM((16,), jnp.float32),
            pltpu.VMEM((BPW,), jnp.float32),
            pltpu.VMEM((13000,), jnp.float32),
            pltpu.VMEM_SHARED((1040000,), jnp.float32),
            pltpu.SemaphoreType.DMA,
            pltpu.SemaphoreType.DMA,
            pltpu.SemaphoreType.DMA,
        ],
    )(_sc_kernel)

    out = run(xt_flat, bias16, tbl_flat)
    return out.reshape(B, 1)


# speedup vs baseline: 1.3151x; 1.0359x over previous
import functools

import jax
import jax.numpy as jnp
from jax import lax
from jax.experimental import pallas as pl
from jax.experimental.pallas import tpu as pltpu
from jax.experimental.pallas import tpu_sc as plsc

B = 16384
F = 26
NC = 2
NS = 16
NW = NC * NS
BPW = B // NW
CHUNK = BPW * F
FIELD = 40000


def _sc_kernel(xt_hbm, bias_hbm, tbl_hbm, out_hbm,
               idx_v, vals_v, bias_v, obuf_v, tstage_v, tbl_sp,
               sem, tsem, gsem):
    sid = lax.axis_index("s")
    wid = lax.axis_index("c") * NS + sid

    tpw = 1040000 // NS
    h1, h2 = 32504, 32496
    tbase = sid * tpw
    tdesc = pltpu.async_copy(tbl_hbm.at[pl.ds(tbase, h1)],
                             tstage_v.at[pl.ds(0, h1)], tsem)
    descs = [
        pltpu.async_copy(
            xt_hbm.at[pl.ds(f * B + wid * BPW, BPW)],
            idx_v.at[pl.ds(f * BPW, BPW)],
            sem,
        )
        for f in range(F)
    ]
    pltpu.sync_copy(bias_hbm, bias_v)
    tdesc.wait()
    pltpu.sync_copy(tstage_v.at[pl.ds(0, h1)], tbl_sp.at[pl.ds(tbase, h1)])
    pltpu.sync_copy(tbl_hbm.at[pl.ds(tbase + h1, h2)],
                    tstage_v.at[pl.ds(0, h2)])
    pltpu.sync_copy(tstage_v.at[pl.ds(0, h2)],
                    tbl_sp.at[pl.ds(tbase + h1, h2)])
    for d in descs:
        d.wait()
    plsc.subcore_barrier()

    pltpu.async_copy(tbl_sp.at[idx_v], vals_v, gsem).wait()

    bias16 = bias_v[...]

    def reduce16(c, _):
        r = c * 16
        acc = vals_v[pl.ds(r, 16)]
        for f in range(1, F):
            acc = acc + vals_v[pl.ds(f * BPW + r, 16)]
        obuf_v[pl.ds(r, 16)] = acc + bias16
        return _
    lax.fori_loop(0, BPW // 16, reduce16, 0)

    pltpu.sync_copy(obuf_v, out_hbm.at[pl.ds(wid * BPW, BPW)])


@jax.jit
def kernel(x, table, bias):
    offsets = jnp.arange(F, dtype=jnp.int32) * FIELD
    xt_flat = (x.astype(jnp.int32) + offsets[None, :]).T.reshape(-1)
    tbl_flat = table.reshape(-1)
    bias16 = jnp.broadcast_to(bias.astype(jnp.float32), (16,))

    run = functools.partial(
        pl.kernel,
        mesh=plsc.VectorSubcoreMesh(core_axis_name="c", subcore_axis_name="s"),
        out_type=jax.ShapeDtypeStruct((B,), jnp.float32),
        compiler_params=pltpu.CompilerParams(needs_layout_passes=False),
        scratch_types=[
            pltpu.VMEM((CHUNK,), jnp.int32),
            pltpu.VMEM((CHUNK,), jnp.float32),
            pltpu.VMEM((16,), jnp.float32),
            pltpu.VMEM((BPW,), jnp.float32),
            pltpu.VMEM((32504,), jnp.float32),
            pltpu.VMEM_SHARED((1040000,), jnp.float32),
            pltpu.SemaphoreType.DMA,
            pltpu.SemaphoreType.DMA,
            pltpu.SemaphoreType.DMA,
        ],
    )(_sc_kernel)

    out = run(xt_flat, bias16, tbl_flat)
    return out.reshape(B, 1)


# speedup vs baseline: 1.3217x; 1.0050x over previous
import functools

import jax
import jax.numpy as jnp
from jax import lax
from jax.experimental import pallas as pl
from jax.experimental.pallas import tpu as pltpu
from jax.experimental.pallas import tpu_sc as plsc

B = 16384
F = 26
NC = 2
NS = 16
NW = NC * NS
BPW = B // NW
CHUNK = BPW * F
FIELD = 40000


def _sc_kernel(xt_hbm, bias_hbm, tbl_hbm, out_hbm,
               idx_v, vals_v, bias_v, obuf_v, tstage_v, tstage2_v, tbl_sp,
               sem, tsem, ssem, gsem, gsem2):
    sid = lax.axis_index("s")
    wid = lax.axis_index("c") * NS + sid

    tpw = 1040000 // NS
    tbase = sid * tpw
    sizes = (16248, 16248, 16248, 16256)
    offs = (0, 16248, 32496, 48744)
    bufs = (tstage_v, tstage2_v)
    fetch = [None] * 4
    store = [None] * 4
    fetch[0] = pltpu.async_copy(tbl_hbm.at[pl.ds(tbase, sizes[0])],
                                bufs[0].at[pl.ds(0, sizes[0])], tsem)
    descs = [
        pltpu.async_copy(
            xt_hbm.at[pl.ds(f * B + wid * BPW, BPW)],
            idx_v.at[pl.ds(f * BPW, BPW)],
            sem,
        )
        for f in range(F)
    ]
    pltpu.sync_copy(bias_hbm, bias_v)
    for k in range(4):
        fetch[k].wait()
        store[k] = pltpu.async_copy(
            bufs[k % 2].at[pl.ds(0, sizes[k])],
            tbl_sp.at[pl.ds(tbase + offs[k], sizes[k])], ssem)
        if k < 3:
            if k >= 1:
                store[k - 1].wait()
            fetch[k + 1] = pltpu.async_copy(
                tbl_hbm.at[pl.ds(tbase + offs[k + 1], sizes[k + 1])],
                bufs[(k + 1) % 2].at[pl.ds(0, sizes[k + 1])], tsem)
    store[2].wait()
    store[3].wait()
    for d in descs:
        d.wait()
    plsc.subcore_barrier()

    HALF = (F // 2) * BPW
    g1 = pltpu.async_copy(tbl_sp.at[idx_v.at[pl.ds(0, HALF)]],
                          vals_v.at[pl.ds(0, HALF)], gsem)
    g2 = pltpu.async_copy(tbl_sp.at[idx_v.at[pl.ds(HALF, CHUNK - HALF)]],
                          vals_v.at[pl.ds(HALF, CHUNK - HALF)], gsem2)
    bias16 = bias_v[...]
    g1.wait()

    def reduce_lo(c, _):
        r = c * 16
        acc = vals_v[pl.ds(r, 16)]
        for f in range(1, F // 2):
            acc = acc + vals_v[pl.ds(f * BPW + r, 16)]
        obuf_v[pl.ds(r, 16)] = acc + bias16
        return _
    lax.fori_loop(0, BPW // 16, reduce_lo, 0)
    g2.wait()

    def reduce_hi(c, _):
        r = c * 16
        acc = obuf_v[pl.ds(r, 16)]
        for f in range(F // 2, F):
            acc = acc + vals_v[pl.ds(f * BPW + r, 16)]
        obuf_v[pl.ds(r, 16)] = acc
        return _
    lax.fori_loop(0, BPW // 16, reduce_hi, 0)

    pltpu.sync_copy(obuf_v, out_hbm.at[pl.ds(wid * BPW, BPW)])


@jax.jit
def kernel(x, table, bias):
    offsets = jnp.arange(F, dtype=jnp.int32) * FIELD
    xt_flat = (x.astype(jnp.int32) + offsets[None, :]).T.reshape(-1)
    tbl_flat = table.reshape(-1)
    bias16 = jnp.broadcast_to(bias.astype(jnp.float32), (16,))

    run = functools.partial(
        pl.kernel,
        mesh=plsc.VectorSubcoreMesh(core_axis_name="c", subcore_axis_name="s"),
        out_type=jax.ShapeDtypeStruct((B,), jnp.float32),
        compiler_params=pltpu.CompilerParams(needs_layout_passes=False),
        scratch_types=[
            pltpu.VMEM((CHUNK,), jnp.int32),
            pltpu.VMEM((CHUNK,), jnp.float32),
            pltpu.VMEM((16,), jnp.float32),
            pltpu.VMEM((BPW,), jnp.float32),
            pltpu.VMEM((16256,), jnp.float32),
            pltpu.VMEM((16256,), jnp.float32),
            pltpu.VMEM_SHARED((1040000,), jnp.float32),
            pltpu.SemaphoreType.DMA,
            pltpu.SemaphoreType.DMA,
            pltpu.SemaphoreType.DMA,
            pltpu.SemaphoreType.DMA,
            pltpu.SemaphoreType.DMA,
        ],
    )(_sc_kernel)

    out = run(xt_flat, bias16, tbl_flat)
    return out.reshape(B, 1)


# speedup vs baseline: 1.3273x; 1.0042x over previous
import functools

import jax
import jax.numpy as jnp
from jax import lax
from jax.experimental import pallas as pl
from jax.experimental.pallas import tpu as pltpu
from jax.experimental.pallas import tpu_sc as plsc

B = 16384
F = 26
NC = 2
NS = 16
NW = NC * NS
BPW = B // NW
CHUNK = BPW * F
FIELD = 40000


def _sc_kernel(xt_hbm, bias_hbm, tbl_hbm, out_hbm,
               xbuf_v, idx_v, vals_v, bias_v, obuf_v, tstage_v, tstage2_v,
               tbl_sp, sem, tsem, ssem, gsem, gsem2):
    sid = lax.axis_index("s")
    wid = lax.axis_index("c") * NS + sid

    tpw = 1040000 // NS
    tbase = sid * tpw
    sizes = (11000, 11000, 11000, 11000, 11000, 10000)
    offs = (0, 11000, 22000, 33000, 44000, 55000)
    NH = len(sizes)
    bufs = (tstage_v, tstage2_v)
    fetch = [None] * NH
    store = [None] * NH
    fetch[0] = pltpu.async_copy(tbl_hbm.at[pl.ds(tbase, sizes[0])],
                                bufs[0].at[pl.ds(0, sizes[0])], tsem)
    xdesc = pltpu.async_copy(xt_hbm.at[:, pl.ds(wid * BPW, BPW)], xbuf_v, sem)
    pltpu.sync_copy(bias_hbm, bias_v)
    for k in range(NH):
        fetch[k].wait()
        store[k] = pltpu.async_copy(
            bufs[k % 2].at[pl.ds(0, sizes[k])],
            tbl_sp.at[pl.ds(tbase + offs[k], sizes[k])], ssem)
        if k < NH - 1:
            if k >= 1:
                store[k - 1].wait()
            fetch[k + 1] = pltpu.async_copy(
                tbl_hbm.at[pl.ds(tbase + offs[k + 1], sizes[k + 1])],
                bufs[(k + 1) % 2].at[pl.ds(0, sizes[k + 1])], tsem)
    xdesc.wait()

    def build_idx(c, _):
        r = c * 16
        for f in range(F):
            idx_v[pl.ds(f * BPW + r, 16)] = xbuf_v[f, pl.ds(r, 16)] + (f * FIELD)
        return _
    lax.fori_loop(0, BPW // 16, build_idx, 0)

    store[NH - 2].wait()
    store[NH - 1].wait()
    plsc.subcore_barrier()

    HALF = (F // 2) * BPW
    g1 = pltpu.async_copy(tbl_sp.at[idx_v.at[pl.ds(0, HALF)]],
                          vals_v.at[pl.ds(0, HALF)], gsem)
    g2 = pltpu.async_copy(tbl_sp.at[idx_v.at[pl.ds(HALF, CHUNK - HALF)]],
                          vals_v.at[pl.ds(HALF, CHUNK - HALF)], gsem2)
    bias16 = bias_v[...]
    g1.wait()

    def reduce_lo(c, _):
        r = c * 16
        acc = vals_v[pl.ds(r, 16)]
        for f in range(1, F // 2):
            acc = acc + vals_v[pl.ds(f * BPW + r, 16)]
        obuf_v[pl.ds(r, 16)] = acc + bias16
        return _
    lax.fori_loop(0, BPW // 16, reduce_lo, 0)
    g2.wait()

    def reduce_hi(c, _):
        r = c * 16
        acc = obuf_v[pl.ds(r, 16)]
        for f in range(F // 2, F):
            acc = acc + vals_v[pl.ds(f * BPW + r, 16)]
        obuf_v[pl.ds(r, 16)] = acc
        return _
    lax.fori_loop(0, BPW // 16, reduce_hi, 0)

    pltpu.sync_copy(obuf_v, out_hbm.at[pl.ds(wid * BPW, BPW)])


@jax.jit
def kernel(x, table, bias):
    xt = x.astype(jnp.int32).T
    tbl_flat = table.reshape(-1)
    bias16 = jnp.broadcast_to(bias.astype(jnp.float32), (16,))

    run = functools.partial(
        pl.kernel,
        mesh=plsc.VectorSubcoreMesh(core_axis_name="c", subcore_axis_name="s"),
        out_type=jax.ShapeDtypeStruct((B,), jnp.float32),
        compiler_params=pltpu.CompilerParams(needs_layout_passes=False),
        scratch_types=[
            pltpu.VMEM((F, BPW), jnp.int32),
            pltpu.VMEM((CHUNK,), jnp.int32),
            pltpu.VMEM((CHUNK,), jnp.float32),
            pltpu.VMEM((16,), jnp.float32),
            pltpu.VMEM((BPW,), jnp.float32),
            pltpu.VMEM((11000,), jnp.float32),
            pltpu.VMEM((11000,), jnp.float32),
            pltpu.VMEM_SHARED((1040000,), jnp.float32),
            pltpu.SemaphoreType.DMA,
            pltpu.SemaphoreType.DMA,
            pltpu.SemaphoreType.DMA,
            pltpu.SemaphoreType.DMA,
            pltpu.SemaphoreType.DMA,
        ],
    )(_sc_kernel)

    out = run(xt, bias16, tbl_flat)
    return out.reshape(B, 1)
